# Initial kernel scaffold; baseline (speedup 1.0000x reference)
import jax
import jax.numpy as jnp
from jax.experimental import pallas as pl


def kernel(x, edge_index, batch, W1, as1, ad1, b1, W2, as2, ad2, b2, W3, as3, ad3, b3, Wr, br, Wm0, bm0, Wm1, bm1, Wl, bl):
    raise NotImplementedError("write your pallas kernel here")


# speedup vs baseline: 66.6602x; 66.6602x over previous
import functools

import jax
import jax.numpy as jnp
from jax import lax
from jax.experimental import pallas as pl
from jax.experimental.pallas import tpu as pltpu
from jax.experimental.pallas import tpu_sc as plsc

N = 10000
E = 320000
D = 128
H = 32
G = 64
FH = 12
TV = 4

NC = 2
NS = 16
NW = NC * NS
L = 16

EA = E + N
CH = 128
NCH = 81
EPT = CH * NCH
EPAD = EPT * NW
NPAD = 10240
RPT = NPAD // NS
HRPT = NPAD // NS

_f32 = jnp.float32


def _dense1_body(x_ref, w_ref, as_ref, ad_ref, h_ref, als_ref, ald_ref):
    h = jnp.dot(x_ref[...], w_ref[...], preferred_element_type=_f32)
    h_ref[pl.ds(0, N), :] = h
    h_ref[pl.ds(N, NPAD - N), :] = jnp.zeros((NPAD - N, H), _f32)
    als_ref[...] = jnp.sum(h * as_ref[...][None, :], axis=1)
    ald_ref[...] = jnp.sum(h * ad_ref[...][None, :], axis=1)


def _dense1(x, w, a_s, a_d):
    return pl.pallas_call(
        _dense1_body,
        out_shape=(
            jax.ShapeDtypeStruct((NPAD, H), _f32),
            jax.ShapeDtypeStruct((N,), _f32),
            jax.ShapeDtypeStruct((N,), _f32),
        ),
    )(x, w, a_s, a_d)


def _combine_body(u_ref, s_ref, b_ref, w_ref, as_ref, ad_ref,
                  h_ref, als_ref, ald_ref):
    u = u_ref[0, :N, :] + u_ref[1, :N, :]
    s = s_ref[0, :N] + s_ref[1, :N]
    out = u / (s + 1e-16)[:, None] + b_ref[...][None, :]
    hg = 0.5 * out * (1.0 + lax.erf(out * jnp.float32(0.7071067811865476)))
    h = jnp.dot(hg, w_ref[...], preferred_element_type=_f32)
    h_ref[pl.ds(0, N), :] = h
    h_ref[pl.ds(N, NPAD - N), :] = jnp.zeros((NPAD - N, H), _f32)
    als_ref[...] = jnp.sum(h * as_ref[...][None, :], axis=1)
    ald_ref[...] = jnp.sum(h * ad_ref[...][None, :], axis=1)


def _combine(u, s, b, w, a_s, a_d):
    return pl.pallas_call(
        _combine_body,
        out_shape=(
            jax.ShapeDtypeStruct((NPAD, H), _f32),
            jax.ShapeDtypeStruct((N,), _f32),
            jax.ShapeDtypeStruct((N,), _f32),
        ),
    )(u, s, b, w, a_s, a_d)


def _head_body(u_ref, s_ref, b_ref, batch_ref,
               wr_ref, br_ref, wm0_ref, bm0_ref, wm1_ref, bm1_ref,
               wl_ref, bl_ref, z_ref, xmax_ref):
    u = u_ref[0, :N, :] + u_ref[1, :N, :]
    s = s_ref[0, :N] + s_ref[1, :N]
    h = u / (s + 1e-16)[:, None] + b_ref[...][None, :]

    batch = batch_ref[...]
    gids = lax.broadcasted_iota(jnp.int32, (G, N), 0)
    onehot = (gids == batch[None, :]).astype(_f32)
    cnt = jnp.sum(onehot, axis=1)
    xsum = jnp.dot(onehot, h, preferred_element_type=_f32)
    xmean = xsum / jnp.maximum(cnt, 1.0)[:, None]

    def gmax(g, _):
        mwt = jnp.where(batch == g, 0.0, -jnp.inf).astype(_f32)
        m = jnp.max(h + mwt[:, None], axis=0)
        xmax_ref[pl.ds(g, 1), :] = m[None, :]
        return 0

    lax.fori_loop(0, G, gmax, 0)
    xmax = xmax_ref[...]
    xmax = jnp.where(jnp.isfinite(xmax), xmax, 0.0)

    z = jnp.concatenate([xmean, xmax], axis=1)
    z = jnp.dot(z, wr_ref[...], preferred_element_type=_f32) + br_ref[...][None, :]
    z = jnp.maximum(jnp.dot(z, wm0_ref[...], preferred_element_type=_f32)
                    + bm0_ref[...][None, :], 0.0)
    z = jnp.maximum(jnp.dot(z, wm1_ref[...], preferred_element_type=_f32)
                    + bm1_ref[...][None, :], 0.0)
    z_ref[...] = (jnp.dot(z, wl_ref[...], preferred_element_type=_f32)
                  + bl_ref[...][None, :])


def _head(u, s, b, batch, wr, br, wm0, bm0, wm1, bm1, wl, bl):
    return pl.pallas_call(
        _head_body,
        out_shape=jax.ShapeDtypeStruct((G, TV * FH), _f32),
        scratch_shapes=[pltpu.VMEM((G, H), _f32)],
    )(u, s, b, batch, wr, br, wm0, bm0, wm1, bm1, wl, bl)


def _edge_body(src_hbm, dst_hbm, h_hbm, als_hbm, ald_hbm,
               u_out, s_out,
               src_v, dst_v, als_v, ald_v, ex_v, rows_v, zs_v, hst_v,
               sh_h, sh_u, sh_s, sem):
    cid = lax.axis_index("c")
    sid = lax.axis_index("s")
    wid = sid * NC + cid

    pltpu.sync_copy(src_hbm.at[wid], src_v)
    pltpu.sync_copy(dst_hbm.at[wid], dst_v)
    pltpu.sync_copy(als_hbm, als_v)
    pltpu.sync_copy(ald_hbm, ald_v)
    pltpu.sync_copy(h_hbm.at[pl.ds(sid * HRPT, HRPT)], hst_v)
    pltpu.sync_copy(hst_v, sh_h.at[pl.ds(sid * HRPT, HRPT)])

    zero16 = jnp.zeros((L,), _f32)

    def zrow(k, _):
        rows_v[k, pl.ds(0, L)] = zero16
        rows_v[k, pl.ds(L, L)] = zero16
        return 0

    lax.fori_loop(0, CH, zrow, 0)

    def zs(k, _):
        zs_v[pl.ds(k * L, L)] = zero16
        return 0

    lax.fori_loop(0, RPT // L, zs, 0)

    for r in range(RPT // CH):
        pltpu.sync_copy(rows_v, sh_u.at[pl.ds(sid * RPT + r * CH, CH)])
    pltpu.sync_copy(zs_v, sh_s.at[pl.ds(sid * RPT, RPT)])

    plsc.subcore_barrier()

    edge_base = wid * EPT

    def chunk(j, _):
        for g in range(CH // L):
            off = g * L
            s16 = src_v[j, pl.ds(off, L)]
            d16 = dst_v[j, pl.ds(off, L)]
            e = (plsc.load_gather(als_v, [s16])
                 + plsc.load_gather(ald_v, [d16]))
            e = jnp.where(e >= 0.0, e, e * 0.2)
            gid = edge_base + j * CH + off + lax.iota(jnp.int32, L)
            ex_v[j, pl.ds(off, L)] = jnp.where(gid < EA, jnp.exp(e), 0.0)

        pltpu.sync_copy(sh_h.at[src_v.at[j]], rows_v)

        def scale(g, _):
            ex16 = ex_v[j, pl.ds(g * L, L)]
            for k in range(L):
                r = g * L + k
                exk = ex16[k]
                rows_v[r, pl.ds(0, L)] = rows_v[r, pl.ds(0, L)] * exk
                rows_v[r, pl.ds(L, L)] = rows_v[r, pl.ds(L, L)] * exk
            return 0

        lax.fori_loop(0, CH // L, scale, 0)

        pltpu.sync_copy(rows_v, sh_u.at[dst_v.at[j]], add=True)
        pltpu.sync_copy(ex_v.at[j], sh_s.at[dst_v.at[j]], add=True)
        return 0

    lax.fori_loop(0, NCH, chunk, 0)

    plsc.subcore_barrier()

    pltpu.sync_copy(sh_u.at[pl.ds(sid * RPT, RPT)], hst_v)
    pltpu.sync_copy(hst_v, u_out.at[cid, pl.ds(sid * RPT, RPT)])
    pltpu.sync_copy(sh_s.at[pl.ds(sid * RPT, RPT)], zs_v)
    pltpu.sync_copy(zs_v, s_out.at[cid, pl.ds(sid * RPT, RPT)])


@functools.cache
def _edge_pass_fn():
    return pl.kernel(
        _edge_body,
        out_type=(
            jax.ShapeDtypeStruct((NC, NPAD, H), _f32),
            jax.ShapeDtypeStruct((NC, NPAD), _f32),
        ),
        mesh=plsc.VectorSubcoreMesh(core_axis_name="c", subcore_axis_name="s",
                                    num_cores=NC, num_subcores=NS),
        compiler_params=pltpu.CompilerParams(needs_layout_passes=False, use_tc_tiling_on_sc=False),
        scratch_types=[
            pltpu.VMEM((NCH, CH), jnp.int32),
            pltpu.VMEM((NCH, CH), jnp.int32),
            pltpu.VMEM((N,), _f32),
            pltpu.VMEM((N,), _f32),
            pltpu.VMEM((NCH, CH), _f32),
            pltpu.VMEM((CH, H), _f32),
            pltpu.VMEM((RPT,), _f32),
            pltpu.VMEM((RPT, H), _f32),
            pltpu.VMEM_SHARED((NPAD, H), _f32),
            pltpu.VMEM_SHARED((NPAD, H), _f32),
            pltpu.VMEM_SHARED((NPAD,), _f32),
            pltpu.SemaphoreType.DMA,
        ],
    )


def _edge_pass(src, dst, h, als, ald):
    return _edge_pass_fn()(src, dst, h, als, ald)


def kernel(x, edge_index, batch, W1, as1, ad1, b1, W2, as2, ad2, b2,
           W3, as3, ad3, b3, Wr, br, Wm0, bm0, Wm1, bm1, Wl, bl):
    loops = jnp.arange(N, dtype=jnp.int32)
    pad = jnp.zeros((EPAD - EA,), jnp.int32)
    src = jnp.concatenate([edge_index[0], loops, pad]).reshape(NW, NCH, CH)
    dst = jnp.concatenate([edge_index[1], loops, pad]).reshape(NW, NCH, CH)

    h, als, ald = _dense1(x, W1, as1, ad1)
    u, s = _edge_pass(src, dst, h, als, ald)
    h, als, ald = _combine(u, s, b1, W2, as2, ad2)
    u, s = _edge_pass(src, dst, h, als, ald)
    h, als, ald = _combine(u, s, b2, W3, as3, ad3)
    u, s = _edge_pass(src, dst, h, als, ald)
    z = _head(u, s, b3, batch, Wr, br, Wm0, bm0, Wm1, bm1, Wl, bl)
    return z.reshape(G, FH, TV)


# speedup vs baseline: 74.6185x; 1.1194x over previous
import functools

import jax
import jax.numpy as jnp
from jax import lax
from jax.experimental import pallas as pl
from jax.experimental.pallas import tpu as pltpu
from jax.experimental.pallas import tpu_sc as plsc

N = 10000
E = 320000
D = 128
H = 32
G = 64
FH = 12
TV = 4

NC = 2
NS = 16
NW = NC * NS
L = 16

EA = E + N
CH = 128
NCH = 82
EPT = CH * NCH
EPAD = EPT * NW
NPAD = 10240
RPT = NPAD // NS
HRPT = NPAD // NS

_f32 = jnp.float32


def _dense1_body(x_ref, w_ref, as_ref, ad_ref, h_ref, als_ref, ald_ref):
    h = jnp.dot(x_ref[...], w_ref[...], preferred_element_type=_f32)
    h_ref[pl.ds(0, N), :] = h
    h_ref[pl.ds(N, NPAD - N), :] = jnp.zeros((NPAD - N, H), _f32)
    als_ref[...] = jnp.sum(h * as_ref[...][None, :], axis=1)
    ald_ref[...] = jnp.sum(h * ad_ref[...][None, :], axis=1)


def _dense1(x, w, a_s, a_d):
    return pl.pallas_call(
        _dense1_body,
        out_shape=(
            jax.ShapeDtypeStruct((NPAD, H), _f32),
            jax.ShapeDtypeStruct((N,), _f32),
            jax.ShapeDtypeStruct((N,), _f32),
        ),
    )(x, w, a_s, a_d)


def _combine_body(u_ref, s_ref, b_ref, w_ref, as_ref, ad_ref,
                  h_ref, als_ref, ald_ref):
    u = u_ref[0, :N, :] + u_ref[1, :N, :]
    s = s_ref[0, :N] + s_ref[1, :N]
    out = u / (s + 1e-16)[:, None] + b_ref[...][None, :]
    hg = 0.5 * out * (1.0 + lax.erf(out * jnp.float32(0.7071067811865476)))
    h = jnp.dot(hg, w_ref[...], preferred_element_type=_f32)
    h_ref[pl.ds(0, N), :] = h
    h_ref[pl.ds(N, NPAD - N), :] = jnp.zeros((NPAD - N, H), _f32)
    als_ref[...] = jnp.sum(h * as_ref[...][None, :], axis=1)
    ald_ref[...] = jnp.sum(h * ad_ref[...][None, :], axis=1)


def _combine(u, s, b, w, a_s, a_d):
    return pl.pallas_call(
        _combine_body,
        out_shape=(
            jax.ShapeDtypeStruct((NPAD, H), _f32),
            jax.ShapeDtypeStruct((N,), _f32),
            jax.ShapeDtypeStruct((N,), _f32),
        ),
    )(u, s, b, w, a_s, a_d)


def _head_body(u_ref, s_ref, b_ref, batch_ref,
               wr_ref, br_ref, wm0_ref, bm0_ref, wm1_ref, bm1_ref,
               wl_ref, bl_ref, z_ref, xmax_ref):
    u = u_ref[0, :N, :] + u_ref[1, :N, :]
    s = s_ref[0, :N] + s_ref[1, :N]
    h = u / (s + 1e-16)[:, None] + b_ref[...][None, :]

    batch = batch_ref[...]
    gids = lax.broadcasted_iota(jnp.int32, (G, N), 0)
    onehot = (gids == batch[None, :]).astype(_f32)
    cnt = jnp.sum(onehot, axis=1)
    xsum = jnp.dot(onehot, h, preferred_element_type=_f32)
    xmean = xsum / jnp.maximum(cnt, 1.0)[:, None]

    def gmax(g, _):
        mwt = jnp.where(batch == g, 0.0, -jnp.inf).astype(_f32)
        m = jnp.max(h + mwt[:, None], axis=0)
        xmax_ref[pl.ds(g, 1), :] = m[None, :]
        return 0

    lax.fori_loop(0, G, gmax, 0)
    xmax = xmax_ref[...]
    xmax = jnp.where(jnp.isfinite(xmax), xmax, 0.0)

    z = jnp.concatenate([xmean, xmax], axis=1)
    z = jnp.dot(z, wr_ref[...], preferred_element_type=_f32) + br_ref[...][None, :]
    z = jnp.maximum(jnp.dot(z, wm0_ref[...], preferred_element_type=_f32)
                    + bm0_ref[...][None, :], 0.0)
    z = jnp.maximum(jnp.dot(z, wm1_ref[...], preferred_element_type=_f32)
                    + bm1_ref[...][None, :], 0.0)
    z_ref[...] = (jnp.dot(z, wl_ref[...], preferred_element_type=_f32)
                  + bl_ref[...][None, :])


def _head(u, s, b, batch, wr, br, wm0, bm0, wm1, bm1, wl, bl):
    return pl.pallas_call(
        _head_body,
        out_shape=jax.ShapeDtypeStruct((G, TV * FH), _f32),
        scratch_shapes=[pltpu.VMEM((G, H), _f32)],
    )(u, s, b, batch, wr, br, wm0, bm0, wm1, bm1, wl, bl)


def _edge_body(src_hbm, dst_hbm, h_hbm, als_hbm, ald_hbm,
               u_out, s_out,
               src_v, dst_v, als_v, ald_v, ex_v, rows_v, rows2_v, zs_v, hst_v,
               sh_h, sh_u, sh_s, sem):
    cid = lax.axis_index("c")
    sid = lax.axis_index("s")
    wid = sid * NC + cid

    pltpu.sync_copy(src_hbm.at[wid], src_v)
    pltpu.sync_copy(dst_hbm.at[wid], dst_v)
    pltpu.sync_copy(als_hbm, als_v)
    pltpu.sync_copy(ald_hbm, ald_v)
    pltpu.sync_copy(h_hbm.at[pl.ds(sid * HRPT, HRPT)], hst_v)
    pltpu.sync_copy(hst_v, sh_h.at[pl.ds(sid * HRPT, HRPT)])

    zero16 = jnp.zeros((L,), _f32)

    def zrow(k, _):
        rows_v[k, pl.ds(0, L)] = zero16
        rows_v[k, pl.ds(L, L)] = zero16
        return 0

    lax.fori_loop(0, CH, zrow, 0)

    def zs(k, _):
        zs_v[pl.ds(k * L, L)] = zero16
        return 0

    lax.fori_loop(0, RPT // L, zs, 0)

    for r in range(RPT // CH):
        pltpu.sync_copy(rows_v, sh_u.at[pl.ds(sid * RPT + r * CH, CH)])
    pltpu.sync_copy(zs_v, sh_s.at[pl.ds(sid * RPT, RPT)])

    plsc.subcore_barrier()

    edge_base = wid * EPT
    bufs = (rows_v, rows2_v)

    pltpu.async_copy(sh_h.at[src_v.at[0]], rows_v, sem)

    def chunk2(j2, _):
        for b in range(2):
            j = j2 * 2 + b
            buf = bufs[b]
            nbuf = bufs[1 - b]
            pltpu.make_async_copy(sh_h.at[src_v.at[j]], buf, sem).wait()

            @pl.when(j < NCH - 1)
            def _():
                pltpu.async_copy(sh_h.at[src_v.at[j + 1]], nbuf, sem)

            for g in range(CH // L):
                off = g * L
                s16 = src_v[j, pl.ds(off, L)]
                d16 = dst_v[j, pl.ds(off, L)]
                e = (plsc.load_gather(als_v, [s16])
                     + plsc.load_gather(ald_v, [d16]))
                e = jnp.where(e >= 0.0, e, e * 0.2)
                gid = edge_base + j * CH + off + lax.iota(jnp.int32, L)
                ex_v[j, pl.ds(off, L)] = jnp.where(gid < EA, jnp.exp(e), 0.0)

            def scale(g, _):
                ex16 = ex_v[j, pl.ds(g * L, L)]
                for k in range(L):
                    r = g * L + k
                    exk = ex16[k]
                    buf[r, pl.ds(0, L)] = buf[r, pl.ds(0, L)] * exk
                    buf[r, pl.ds(L, L)] = buf[r, pl.ds(L, L)] * exk
                return 0

            lax.fori_loop(0, CH // L, scale, 0)

            pltpu.sync_copy(buf, sh_u.at[dst_v.at[j]], add=True)
            pltpu.sync_copy(ex_v.at[j], sh_s.at[dst_v.at[j]], add=True)
        return 0

    lax.fori_loop(0, NCH // 2, chunk2, 0)

    plsc.subcore_barrier()

    pltpu.sync_copy(sh_u.at[pl.ds(sid * RPT, RPT)], hst_v)
    pltpu.sync_copy(hst_v, u_out.at[cid, pl.ds(sid * RPT, RPT)])
    pltpu.sync_copy(sh_s.at[pl.ds(sid * RPT, RPT)], zs_v)
    pltpu.sync_copy(zs_v, s_out.at[cid, pl.ds(sid * RPT, RPT)])


@functools.cache
def _edge_pass_fn():
    return pl.kernel(
        _edge_body,
        out_type=(
            jax.ShapeDtypeStruct((NC, NPAD, H), _f32),
            jax.ShapeDtypeStruct((NC, NPAD), _f32),
        ),
        mesh=plsc.VectorSubcoreMesh(core_axis_name="c", subcore_axis_name="s",
                                    num_cores=NC, num_subcores=NS),
        compiler_params=pltpu.CompilerParams(needs_layout_passes=False, use_tc_tiling_on_sc=False),
        scratch_types=[
            pltpu.VMEM((NCH, CH), jnp.int32),
            pltpu.VMEM((NCH, CH), jnp.int32),
            pltpu.VMEM((N,), _f32),
            pltpu.VMEM((N,), _f32),
            pltpu.VMEM((NCH, CH), _f32),
            pltpu.VMEM((CH, H), _f32),
            pltpu.VMEM((CH, H), _f32),
            pltpu.VMEM((RPT,), _f32),
            pltpu.VMEM((RPT, H), _f32),
            pltpu.VMEM_SHARED((NPAD, H), _f32),
            pltpu.VMEM_SHARED((NPAD, H), _f32),
            pltpu.VMEM_SHARED((NPAD,), _f32),
            pltpu.SemaphoreType.DMA,
        ],
    )


def _edge_pass(src, dst, h, als, ald):
    return _edge_pass_fn()(src, dst, h, als, ald)


def kernel(x, edge_index, batch, W1, as1, ad1, b1, W2, as2, ad2, b2,
           W3, as3, ad3, b3, Wr, br, Wm0, bm0, Wm1, bm1, Wl, bl):
    loops = jnp.arange(N, dtype=jnp.int32)
    pad = jnp.zeros((EPAD - EA,), jnp.int32)
    src = jnp.concatenate([edge_index[0], loops, pad]).reshape(NW, NCH, CH)
    dst = jnp.concatenate([edge_index[1], loops, pad]).reshape(NW, NCH, CH)

    h, als, ald = _dense1(x, W1, as1, ad1)
    u, s = _edge_pass(src, dst, h, als, ald)
    h, als, ald = _combine(u, s, b1, W2, as2, ad2)
    u, s = _edge_pass(src, dst, h, als, ald)
    h, als, ald = _combine(u, s, b2, W3, as3, ad3)
    u, s = _edge_pass(src, dst, h, als, ald)
    z = _head(u, s, b3, batch, Wr, br, Wm0, bm0, Wm1, bm1, Wl, bl)
    return z.reshape(G, FH, TV)
